# baseline (speedup 1.0000x reference)
import functools

import jax
import jax.numpy as jnp
from jax import lax
from jax.experimental import pallas as pl
from jax.experimental.pallas import tpu as pltpu
from jax.experimental.pallas import tpu_sc as plsc

B = 16384
STATE = 32
NC = 2
NS = 16
NW = NC * NS
BPW = B // NW
CH = 128
NCH = BPW // CH
L = 16


def _sc_body(user_hbm, item_hbm, wu_hbm, wi_hbm, y_hbm, out_hbm,
             idx_u, idx_i, rows_u, rows_i, y_v, out_v, tmat, sem):
    c = lax.axis_index("c")
    s = lax.axis_index("s")
    wid = s * NC + c
    base = wid * BPW

    pltpu.sync_copy(user_hbm.at[pl.ds(wid * NCH, NCH)], idx_u)
    pltpu.sync_copy(item_hbm.at[pl.ds(wid * NCH, NCH)], idx_i)
    pltpu.sync_copy(y_hbm, y_v)

    copies = []
    for j in range(NCH):
        copies.append(pltpu.async_copy(
            wu_hbm.at[idx_u.at[j]], rows_u.at[pl.ds(j * CH, CH)], sem))
        copies.append(pltpu.async_copy(
            wi_hbm.at[idx_i.at[j]], rows_i.at[pl.ds(j * CH, CH)], sem))
    for cp in copies:
        cp.wait()

    yu0 = y_v[pl.ds(0, L)]
    yu1 = y_v[pl.ds(L, L)]
    yi0 = y_v[pl.ds(2 * L, L)]
    yi1 = y_v[pl.ds(3 * L, L)]
    lane = lax.iota(jnp.int32, L)

    def chunk_body(ch, carry):
        r0 = ch * L
        for b in range(L):
            r = r0 + b
            m = (rows_u[r, pl.ds(0, L)] * yu0
                 + rows_u[r, pl.ds(L, L)] * yu1
                 + rows_i[r, pl.ds(0, L)] * yi0
                 + rows_i[r, pl.ds(L, L)] * yi1)
            tmat[pl.ds(b * L, L)] = m
        acc = jnp.zeros((L,), jnp.float32)
        for d in range(L):
            acc = acc + plsc.load_gather(tmat, [lane * L + d])
        out_v[pl.ds(r0, L)] = acc
        return carry

    lax.fori_loop(0, BPW // L, chunk_body, 0)
    pltpu.sync_copy(out_v, out_hbm.at[pl.ds(base, BPW)])


def kernel(user, item, W_user, W_item, Y_w):
    mesh = plsc.VectorSubcoreMesh(core_axis_name="c", subcore_axis_name="s")
    k = functools.partial(
        pl.kernel,
        mesh=mesh,
        out_type=jax.ShapeDtypeStruct((B,), jnp.float32),
        scratch_types=[
            pltpu.VMEM((NCH, CH), jnp.int32),
            pltpu.VMEM((NCH, CH), jnp.int32),
            pltpu.VMEM((BPW, STATE), jnp.float32),
            pltpu.VMEM((BPW, STATE), jnp.float32),
            pltpu.VMEM((2 * STATE,), jnp.float32),
            pltpu.VMEM((BPW,), jnp.float32),
            pltpu.VMEM((L * L,), jnp.float32),
            pltpu.SemaphoreType.DMA,
        ],
        compiler_params=pltpu.CompilerParams(
            needs_layout_passes=False, use_tc_tiling_on_sc=False),
    )(_sc_body)
    user2d = user.reshape(B // CH, CH)
    item2d = item.reshape(B // CH, CH)
    y64 = Y_w.reshape(2 * STATE)
    out = k(user2d, item2d, W_user, W_item, y64)
    return out.reshape(B, 1)


# speedup vs baseline: 1.6471x; 1.6471x over previous
import functools

import jax
import jax.numpy as jnp
from jax import lax
from jax.experimental import pallas as pl
from jax.experimental.pallas import tpu as pltpu
from jax.experimental.pallas import tpu_sc as plsc

B = 16384
STATE = 32
NU = 1_000_000
NI = 100_000
NC = 2
NS = 16
NW = NC * NS
BPW = B // NW
CH = 128
NCH = BPW // CH
L = 16

BLK = 2048


def _proj_body(wt_ref, y_ref, p_ref):
    wt = wt_ref[...]
    yv = y_ref[0, :].reshape(STATE, 1)
    p_ref[...] = jnp.sum(wt * yv, axis=0)


def _project(wt, y_col, n):
    grid = (n + BLK - 1) // BLK
    return pl.pallas_call(
        _proj_body,
        grid=(grid,),
        in_specs=[
            pl.BlockSpec((STATE, BLK), lambda g: (0, g)),
            pl.BlockSpec((1, STATE), lambda g: (0, 0)),
        ],
        out_specs=pl.BlockSpec((BLK,), lambda g: (g,)),
        out_shape=jax.ShapeDtypeStruct((n,), jnp.float32),
    )(wt, y_col)


def _gather_body(user_hbm, item_hbm, pu_hbm, pi_hbm, out_hbm,
                 idx_u, idx_i, val_u, val_i, out_v, sem):
    c = lax.axis_index("c")
    s = lax.axis_index("s")
    wid = s * NC + c
    base = wid * BPW

    pltpu.sync_copy(user_hbm.at[pl.ds(wid * NCH, NCH)], idx_u)
    pltpu.sync_copy(item_hbm.at[pl.ds(wid * NCH, NCH)], idx_i)

    copies = []
    for j in range(NCH):
        copies.append(pltpu.async_copy(
            pu_hbm.at[idx_u.at[j]], val_u.at[pl.ds(j * CH, CH)], sem))
        copies.append(pltpu.async_copy(
            pi_hbm.at[idx_i.at[j]], val_i.at[pl.ds(j * CH, CH)], sem))
    for cp in copies:
        cp.wait()

    for j in range(BPW // L):
        out_v[pl.ds(j * L, L)] = (val_u[pl.ds(j * L, L)]
                                  + val_i[pl.ds(j * L, L)])
    pltpu.sync_copy(out_v, out_hbm.at[pl.ds(base, BPW)])


def _gather_add(user2d, item2d, p_u, p_i):
    mesh = plsc.VectorSubcoreMesh(core_axis_name="c", subcore_axis_name="s")
    return functools.partial(
        pl.kernel,
        mesh=mesh,
        out_type=jax.ShapeDtypeStruct((B,), jnp.float32),
        scratch_types=[
            pltpu.VMEM((NCH, CH), jnp.int32),
            pltpu.VMEM((NCH, CH), jnp.int32),
            pltpu.VMEM((BPW,), jnp.float32),
            pltpu.VMEM((BPW,), jnp.float32),
            pltpu.VMEM((BPW,), jnp.float32),
            pltpu.SemaphoreType.DMA,
        ],
        compiler_params=pltpu.CompilerParams(
            needs_layout_passes=False, use_tc_tiling_on_sc=False),
    )(_gather_body)(user2d, item2d, p_u, p_i)


def kernel(user, item, W_user, W_item, Y_w):
    y_u = Y_w[:, :STATE]
    y_i = Y_w[:, STATE:]
    p_u = _project(W_user.T, y_u, NU)
    p_i = _project(W_item.T, y_i, NI)
    out = _gather_add(user.reshape(B // CH, CH), item.reshape(B // CH, CH),
                      p_u, p_i)
    return out.reshape(B, 1)


# speedup vs baseline: 5.6326x; 3.4198x over previous
import functools

import jax
import jax.numpy as jnp
from jax import lax
from jax.experimental import pallas as pl
from jax.experimental.pallas import tpu as pltpu
from jax.experimental.pallas import tpu_sc as plsc

B = 16384
STATE = 32
NU = 1_000_000
NI = 100_000
NC = 2
NS = 16
NW = NC * NS
BPW = B // NW
CH = 128
NCH = BPW // CH
L = 16

BLK = 16384


def _proj_body(wt_ref, y_ref, p_ref):
    wt = wt_ref[...]
    yv = y_ref[0, :].reshape(STATE, 1)
    p_ref[...] = jnp.sum(wt * yv, axis=0)


def _project(wt, y_col, n):
    grid = (n + BLK - 1) // BLK
    return pl.pallas_call(
        _proj_body,
        grid=(grid,),
        in_specs=[
            pl.BlockSpec((STATE, BLK), lambda g: (0, g)),
            pl.BlockSpec((1, STATE), lambda g: (0, 0)),
        ],
        out_specs=pl.BlockSpec((BLK,), lambda g: (g,)),
        out_shape=jax.ShapeDtypeStruct((n,), jnp.float32),
    )(wt, y_col)


def _gather_body(user_hbm, item_hbm, pu_hbm, pi_hbm, out_hbm,
                 idx_u, idx_i, val_u, val_i, out_v, sem):
    c = lax.axis_index("c")
    s = lax.axis_index("s")
    wid = s * NC + c
    base = wid * BPW

    pltpu.sync_copy(user_hbm.at[pl.ds(wid * NCH, NCH)], idx_u)
    pltpu.sync_copy(item_hbm.at[pl.ds(wid * NCH, NCH)], idx_i)

    copies = []
    for j in range(NCH):
        copies.append(pltpu.async_copy(
            pu_hbm.at[idx_u.at[j]], val_u.at[pl.ds(j * CH, CH)], sem))
        copies.append(pltpu.async_copy(
            pi_hbm.at[idx_i.at[j]], val_i.at[pl.ds(j * CH, CH)], sem))
    for cp in copies:
        cp.wait()

    for j in range(BPW // L):
        out_v[pl.ds(j * L, L)] = (val_u[pl.ds(j * L, L)]
                                  + val_i[pl.ds(j * L, L)])
    pltpu.sync_copy(out_v, out_hbm.at[pl.ds(base, BPW)])


def _gather_add(user2d, item2d, p_u, p_i):
    mesh = plsc.VectorSubcoreMesh(core_axis_name="c", subcore_axis_name="s")
    return functools.partial(
        pl.kernel,
        mesh=mesh,
        out_type=jax.ShapeDtypeStruct((B,), jnp.float32),
        scratch_types=[
            pltpu.VMEM((NCH, CH), jnp.int32),
            pltpu.VMEM((NCH, CH), jnp.int32),
            pltpu.VMEM((BPW,), jnp.float32),
            pltpu.VMEM((BPW,), jnp.float32),
            pltpu.VMEM((BPW,), jnp.float32),
            pltpu.SemaphoreType.DMA,
        ],
        compiler_params=pltpu.CompilerParams(
            needs_layout_passes=False, use_tc_tiling_on_sc=False),
    )(_gather_body)(user2d, item2d, p_u, p_i)


def kernel(user, item, W_user, W_item, Y_w):
    y_u = Y_w[:, :STATE]
    y_i = Y_w[:, STATE:]
    p_u = _project(W_user.T, y_u, NU)
    p_i = _project(W_item.T, y_i, NI)
    out = _gather_add(user.reshape(B // CH, CH), item.reshape(B // CH, CH),
                      p_u, p_i)
    return out.reshape(B, 1)


# speedup vs baseline: 6.9203x; 1.2286x over previous
import functools

import jax
import jax.numpy as jnp
from jax import lax
from jax.experimental import pallas as pl
from jax.experimental.pallas import tpu as pltpu
from jax.experimental.pallas import tpu_sc as plsc

B = 16384
STATE = 32
NU = 1_000_000
NI = 100_000
NC = 2
NS = 16
NW = NC * NS
BPW = B // NW
CH = 128
NCH = BPW // CH
L = 16

BLK = 32768


def _proj_body(wt_ref, y_ref, p_ref):
    wt = wt_ref[...]
    yv = y_ref[0, :].reshape(STATE, 1)
    p_ref[...] = jnp.sum(wt * yv, axis=0)


def _project(wt, y_col, n):
    grid = (n + BLK - 1) // BLK
    return pl.pallas_call(
        _proj_body,
        grid=(grid,),
        in_specs=[
            pl.BlockSpec((STATE, BLK), lambda g: (0, g)),
            pl.BlockSpec((1, STATE), lambda g: (0, 0)),
        ],
        out_specs=pl.BlockSpec((BLK,), lambda g: (g,)),
        out_shape=jax.ShapeDtypeStruct((n,), jnp.float32),
    )(wt, y_col)


def _gather_body(user_hbm, item_hbm, pu_hbm, pi_hbm, out_hbm,
                 idx_u, idx_i, val_u, val_i, out_v, sem):
    c = lax.axis_index("c")
    s = lax.axis_index("s")
    wid = s * NC + c
    base = wid * BPW

    pltpu.sync_copy(user_hbm.at[pl.ds(wid * NCH, NCH)], idx_u)
    pltpu.sync_copy(item_hbm.at[pl.ds(wid * NCH, NCH)], idx_i)

    copies = []
    for j in range(NCH):
        copies.append(pltpu.async_copy(
            pu_hbm.at[idx_u.at[j]], val_u.at[pl.ds(j * CH, CH)], sem))
        copies.append(pltpu.async_copy(
            pi_hbm.at[idx_i.at[j]], val_i.at[pl.ds(j * CH, CH)], sem))
    for cp in copies:
        cp.wait()

    for j in range(BPW // L):
        out_v[pl.ds(j * L, L)] = (val_u[pl.ds(j * L, L)]
                                  + val_i[pl.ds(j * L, L)])
    pltpu.sync_copy(out_v, out_hbm.at[pl.ds(base, BPW)])


def _gather_add(user2d, item2d, p_u, p_i):
    mesh = plsc.VectorSubcoreMesh(core_axis_name="c", subcore_axis_name="s")
    return functools.partial(
        pl.kernel,
        mesh=mesh,
        out_type=jax.ShapeDtypeStruct((B,), jnp.float32),
        scratch_types=[
            pltpu.VMEM((NCH, CH), jnp.int32),
            pltpu.VMEM((NCH, CH), jnp.int32),
            pltpu.VMEM((BPW,), jnp.float32),
            pltpu.VMEM((BPW,), jnp.float32),
            pltpu.VMEM((BPW,), jnp.float32),
            pltpu.SemaphoreType.DMA,
        ],
        compiler_params=pltpu.CompilerParams(
            needs_layout_passes=False, use_tc_tiling_on_sc=False),
    )(_gather_body)(user2d, item2d, p_u, p_i)


def kernel(user, item, W_user, W_item, Y_w):
    y_u = Y_w[:, :STATE]
    y_i = Y_w[:, STATE:]
    p_u = _project(W_user.T, y_u, NU)
    p_i = _project(W_item.T, y_i, NI)
    out = _gather_add(user.reshape(B // CH, CH), item.reshape(B // CH, CH),
                      p_u, p_i)
    return out.reshape(B, 1)


# speedup vs baseline: 7.6291x; 1.1024x over previous
import functools

import jax
import jax.numpy as jnp
from jax import lax
from jax.experimental import pallas as pl
from jax.experimental.pallas import tpu as pltpu
from jax.experimental.pallas import tpu_sc as plsc

B = 16384
STATE = 32
NU = 1_000_000
NI = 100_000
NC = 2
NS = 16
NW = NC * NS
BPW = B // NW
CH = 128
NCH = BPW // CH
L = 16

BLK = 65536


def _proj_body(wt_ref, y_ref, p_ref):
    wt = wt_ref[...]
    yv = y_ref[0, :].reshape(STATE, 1)
    p_ref[...] = jnp.sum(wt * yv, axis=0)


def _project(wt, y_col, n):
    grid = (n + BLK - 1) // BLK
    return pl.pallas_call(
        _proj_body,
        grid=(grid,),
        in_specs=[
            pl.BlockSpec((STATE, BLK), lambda g: (0, g)),
            pl.BlockSpec((1, STATE), lambda g: (0, 0)),
        ],
        out_specs=pl.BlockSpec((BLK,), lambda g: (g,)),
        out_shape=jax.ShapeDtypeStruct((n,), jnp.float32),
    )(wt, y_col)


def _gather_body(user_hbm, item_hbm, pu_hbm, pi_hbm, out_hbm,
                 idx_u, idx_i, val_u, val_i, out_v, sem):
    c = lax.axis_index("c")
    s = lax.axis_index("s")
    wid = s * NC + c
    base = wid * BPW

    pltpu.sync_copy(user_hbm.at[pl.ds(wid * NCH, NCH)], idx_u)
    pltpu.sync_copy(item_hbm.at[pl.ds(wid * NCH, NCH)], idx_i)

    copies = []
    for j in range(NCH):
        copies.append(pltpu.async_copy(
            pu_hbm.at[idx_u.at[j]], val_u.at[pl.ds(j * CH, CH)], sem))
        copies.append(pltpu.async_copy(
            pi_hbm.at[idx_i.at[j]], val_i.at[pl.ds(j * CH, CH)], sem))
    for cp in copies:
        cp.wait()

    for j in range(BPW // L):
        out_v[pl.ds(j * L, L)] = (val_u[pl.ds(j * L, L)]
                                  + val_i[pl.ds(j * L, L)])
    pltpu.sync_copy(out_v, out_hbm.at[pl.ds(base, BPW)])


def _gather_add(user2d, item2d, p_u, p_i):
    mesh = plsc.VectorSubcoreMesh(core_axis_name="c", subcore_axis_name="s")
    return functools.partial(
        pl.kernel,
        mesh=mesh,
        out_type=jax.ShapeDtypeStruct((B,), jnp.float32),
        scratch_types=[
            pltpu.VMEM((NCH, CH), jnp.int32),
            pltpu.VMEM((NCH, CH), jnp.int32),
            pltpu.VMEM((BPW,), jnp.float32),
            pltpu.VMEM((BPW,), jnp.float32),
            pltpu.VMEM((BPW,), jnp.float32),
            pltpu.SemaphoreType.DMA,
        ],
        compiler_params=pltpu.CompilerParams(
            needs_layout_passes=False, use_tc_tiling_on_sc=False),
    )(_gather_body)(user2d, item2d, p_u, p_i)


def kernel(user, item, W_user, W_item, Y_w):
    y_u = Y_w[:, :STATE]
    y_i = Y_w[:, STATE:]
    p_u = _project(W_user.T, y_u, NU)
    p_i = _project(W_item.T, y_i, NI)
    out = _gather_add(user.reshape(B // CH, CH), item.reshape(B // CH, CH),
                      p_u, p_i)
    return out.reshape(B, 1)
